# speedup vs baseline: 6.3484x; 6.3484x over previous
import functools

import jax
import jax.numpy as jnp
from jax import lax
from jax.experimental import pallas as pl
from jax.experimental.pallas import tpu as pltpu
from jax.experimental.pallas import tpu_sc as plsc

CHUNK = 128


@functools.lru_cache(maxsize=None)
def _make_gather(num_ids: int, vocab: int, dim: int):
  info = plsc.get_sparse_core_info()
  nc, ns = info.num_cores, info.num_subcores
  nw = nc * ns
  assert num_ids % (nw * CHUNK) == 0
  chunks_per_w = num_ids // (nw * CHUNK)

  mesh = plsc.VectorSubcoreMesh(core_axis_name="c", subcore_axis_name="s")

  @functools.partial(
      pl.kernel,
      mesh=mesh,
      out_type=jax.ShapeDtypeStruct((num_ids, dim), jnp.float32),
      scratch_types=[
          pltpu.VMEM((chunks_per_w, CHUNK), jnp.int32),
          pltpu.VMEM((CHUNK, dim), jnp.float32),
          pltpu.SemaphoreType.DMA,
      ],
  )
  def gather_kernel(ids_hbm, table_hbm, out_hbm, idx_v, rows_v, gsem):
    wid = lax.axis_index("s") * nc + lax.axis_index("c")
    pltpu.sync_copy(ids_hbm.at[pl.ds(wid * chunks_per_w, chunks_per_w)],
                    idx_v)

    def chunk_body(j, carry):
      pltpu.async_copy(table_hbm.at[idx_v.at[j]], rows_v, gsem).wait()
      pltpu.sync_copy(
          rows_v,
          out_hbm.at[pl.ds((wid * chunks_per_w + j) * CHUNK, CHUNK)])
      return carry

    lax.fori_loop(0, chunks_per_w, chunk_body, 0)

  return gather_kernel


def kernel(input_ids, embed_table):
  batch, seq = input_ids.shape
  vocab, dim = embed_table.shape
  num_ids = batch * seq
  ids = input_ids.reshape(num_ids // CHUNK, CHUNK).astype(jnp.int32)
  out = _make_gather(num_ids, vocab, dim)(ids, embed_table)
  return out.reshape(batch, seq, dim)


# speedup vs baseline: 9.2476x; 1.4567x over previous
import functools

import jax
import jax.numpy as jnp
from jax import lax
from jax.experimental import pallas as pl
from jax.experimental.pallas import tpu as pltpu
from jax.experimental.pallas import tpu_sc as plsc

CHUNK = 128
NBUF = 4


@functools.lru_cache(maxsize=None)
def _make_gather(num_ids: int, vocab: int, dim: int):
  info = plsc.get_sparse_core_info()
  nc, ns = info.num_cores, info.num_subcores
  nw = nc * ns
  assert num_ids % (nw * CHUNK) == 0
  chunks_per_w = num_ids // (nw * CHUNK)
  assert chunks_per_w % NBUF == 0

  mesh = plsc.VectorSubcoreMesh(core_axis_name="c", subcore_axis_name="s")

  @functools.partial(
      pl.kernel,
      mesh=mesh,
      out_type=jax.ShapeDtypeStruct((num_ids, dim), jnp.float32),
      scratch_types=[
          pltpu.VMEM((chunks_per_w, CHUNK), jnp.int32),
          pltpu.VMEM((NBUF, CHUNK, dim), jnp.float32),
          pltpu.SemaphoreType.DMA((NBUF,)),
          pltpu.SemaphoreType.DMA((NBUF,)),
      ],
  )
  def gather_kernel(ids_hbm, table_hbm, out_hbm, idx_v, rows_v, gsem, ssem):
    wid = lax.axis_index("s") * nc + lax.axis_index("c")
    base = wid * chunks_per_w
    pltpu.sync_copy(ids_hbm.at[pl.ds(base, chunks_per_w)], idx_v)

    def gather_copy(j, b):
      return pltpu.make_async_copy(
          table_hbm.at[idx_v.at[j]], rows_v.at[b], gsem.at[b])

    def store_copy(j, b):
      return pltpu.make_async_copy(
          rows_v.at[b], out_hbm.at[pl.ds((base + j) * CHUNK, CHUNK)],
          ssem.at[b])

    for b in range(NBUF):
      gather_copy(b, b).start()

    def outer(i, carry):
      g = i * NBUF
      for b in range(NBUF):
        j = g + b
        gather_copy(j, b).wait()
        store_copy(j, b).start()
        store_copy(j, b).wait()

        @pl.when(j + NBUF < chunks_per_w)
        def _():
          gather_copy(j + NBUF, b).start()

      return carry

    lax.fori_loop(0, chunks_per_w // NBUF, outer, 0)

  return gather_kernel


def kernel(input_ids, embed_table):
  batch, seq = input_ids.shape
  vocab, dim = embed_table.shape
  num_ids = batch * seq
  ids = input_ids.reshape(num_ids // CHUNK, CHUNK).astype(jnp.int32)
  out = _make_gather(num_ids, vocab, dim)(ids, embed_table)
  return out.reshape(batch, seq, dim)


# speedup vs baseline: 9.2724x; 1.0027x over previous
import functools

import jax
import jax.numpy as jnp
from jax import lax
from jax.experimental import pallas as pl
from jax.experimental.pallas import tpu as pltpu
from jax.experimental.pallas import tpu_sc as plsc

CHUNK = 128
NBUF = 5


@functools.lru_cache(maxsize=None)
def _make_gather(num_ids: int, vocab: int, dim: int):
  info = plsc.get_sparse_core_info()
  nc, ns = info.num_cores, info.num_subcores
  nw = nc * ns
  assert num_ids % (nw * CHUNK) == 0
  n_chunks = num_ids // (nw * CHUNK)
  assert n_chunks % NBUF == 0

  mesh = plsc.VectorSubcoreMesh(core_axis_name="c", subcore_axis_name="s")

  @functools.partial(
      pl.kernel,
      mesh=mesh,
      out_type=jax.ShapeDtypeStruct((num_ids, dim), jnp.float32),
      scratch_types=[
          pltpu.VMEM((n_chunks, CHUNK), jnp.int32),
          pltpu.VMEM((NBUF, CHUNK, dim), jnp.float32),
          pltpu.SemaphoreType.DMA((NBUF,)),
          pltpu.SemaphoreType.DMA((NBUF,)),
      ],
  )
  def gather_kernel(ids_hbm, table_hbm, out_hbm, idx_v, rows_v, gsem, ssem):
    wid = lax.axis_index("s") * nc + lax.axis_index("c")
    base = wid * n_chunks
    pltpu.sync_copy(ids_hbm.at[pl.ds(base, n_chunks)], idx_v)

    def gather_copy(j, b):
      return pltpu.make_async_copy(
          table_hbm.at[idx_v.at[j]], rows_v.at[b], gsem.at[b])

    def store_copy(j, b):
      return pltpu.make_async_copy(
          rows_v.at[b], out_hbm.at[pl.ds((base + j) * CHUNK, CHUNK)],
          ssem.at[b])

    for b in range(NBUF):
      gather_copy(b, b).start()

    def outer(i, carry):
      g = i * NBUF
      for b in range(NBUF):
        j = g + b
        bp = (b - 1) % NBUF
        gather_copy(j, b).wait()
        store_copy(j, b).start()

        @pl.when(j >= 1)
        def _():
          store_copy(j - 1, bp).wait()

        @pl.when(jnp.logical_and(j >= 1, j < n_chunks + 1 - NBUF))
        def _():
          gather_copy(j - 1 + NBUF, bp).start()

      return carry

    lax.fori_loop(0, n_chunks // NBUF, outer, 0)
    store_copy(n_chunks - 1, (n_chunks - 1) % NBUF).wait()

  return gather_kernel


def kernel(input_ids, embed_table):
  batch, seq = input_ids.shape
  vocab, dim = embed_table.shape
  num_ids = batch * seq
  ids = input_ids.reshape(num_ids // CHUNK, CHUNK).astype(jnp.int32)
  out = _make_gather(num_ids, vocab, dim)(ids, embed_table)
  return out.reshape(batch, seq, dim)
